# baseline (speedup 1.0000x reference)
import functools

import jax
import jax.numpy as jnp
from jax.experimental import pallas as pl
from jax.experimental.pallas import tpu as pltpu

LEAK = 0.6
BIAS = 1.6

BI = 1024


def _esn_block(wr_ref, r_ref, u_ref, o_ref, *, res_dim):
    i = pl.program_id(0)
    col = i * BI
    c = col // res_dim
    wt = wr_ref[...]
    rr = r_ref[:, pl.ds(c * res_dim, res_dim)]
    pre = jax.lax.dot_general(
        rr, wt,
        dimension_numbers=(((1,), (1,)), ((), ())),
        preferred_element_type=jnp.float32,
    )
    pre = pre + u_ref[:, pl.ds(col, BI)] + BIAS
    r_slice = r_ref[:, pl.ds(col, BI)]
    o_ref[:, pl.ds(col, BI)] = LEAK * jnp.tanh(pre) + (1.0 - LEAK) * r_slice


def kernel(proj_vars, res_state, wr):
    seq, chunks, res_dim = proj_vars.shape
    flat = chunks * res_dim
    u = proj_vars.reshape(seq, flat)
    r = res_state.reshape(seq, flat)
    w = wr.reshape(flat, res_dim)
    n = flat // BI

    body = functools.partial(_esn_block, res_dim=res_dim)

    out = pl.pallas_call(
        body,
        grid=(n,),
        in_specs=[
            pl.BlockSpec((BI, res_dim), lambda i: (i, 0)),
            pl.BlockSpec((seq, flat), lambda i: (0, 0)),
            pl.BlockSpec((seq, flat), lambda i: (0, 0)),
        ],
        out_specs=pl.BlockSpec((seq, flat), lambda i: (0, 0)),
        out_shape=jax.ShapeDtypeStruct((seq, flat), jnp.float32),
        compiler_params=pltpu.CompilerParams(
            dimension_semantics=("arbitrary",),
        ),
    )(w, r, u)
    return out.reshape(seq, chunks, res_dim)


# speedup vs baseline: 1.0025x; 1.0025x over previous
import functools

import jax
import jax.numpy as jnp
from jax.experimental import pallas as pl
from jax.experimental.pallas import tpu as pltpu

LEAK = 0.6
BIAS = 1.6

BI = 1024


def _esn_block(wr_ref, r_ref, u_ref, o_ref, *, res_dim):
    i = pl.program_id(0)
    col = i * BI
    c = col // res_dim
    wt = wr_ref[...].astype(jnp.bfloat16)
    rr = r_ref[:, pl.ds(c * res_dim, res_dim)].astype(jnp.bfloat16)
    pre = jax.lax.dot_general(
        rr, wt,
        dimension_numbers=(((1,), (1,)), ((), ())),
        preferred_element_type=jnp.float32,
    )
    pre = pre + u_ref[:, pl.ds(col, BI)] + BIAS
    r_slice = r_ref[:, pl.ds(col, BI)]
    o_ref[:, pl.ds(col, BI)] = LEAK * jnp.tanh(pre) + (1.0 - LEAK) * r_slice


def kernel(proj_vars, res_state, wr):
    seq, chunks, res_dim = proj_vars.shape
    flat = chunks * res_dim
    u = proj_vars.reshape(seq, flat)
    r = res_state.reshape(seq, flat)
    w = wr.reshape(flat, res_dim)
    n = flat // BI

    body = functools.partial(_esn_block, res_dim=res_dim)

    out = pl.pallas_call(
        body,
        grid=(n,),
        in_specs=[
            pl.BlockSpec((BI, res_dim), lambda i: (i, 0)),
            pl.BlockSpec((seq, flat), lambda i: (0, 0)),
            pl.BlockSpec((seq, flat), lambda i: (0, 0)),
        ],
        out_specs=pl.BlockSpec((seq, flat), lambda i: (0, 0)),
        out_shape=jax.ShapeDtypeStruct((seq, flat), jnp.float32),
        compiler_params=pltpu.CompilerParams(
            dimension_semantics=("arbitrary",),
        ),
    )(w, r, u)
    return out.reshape(seq, chunks, res_dim)


# speedup vs baseline: 1.0278x; 1.0252x over previous
import functools

import jax
import jax.numpy as jnp
from jax.experimental import pallas as pl
from jax.experimental.pallas import tpu as pltpu

LEAK = 0.6
BIAS = 1.6

BI = 1024


def _esn_block(wr_ref, r_ref, u_ref, o_ref, *, res_dim):
    i = pl.program_id(0)
    col = i * BI
    c = col // res_dim
    wt = wr_ref[...].astype(jnp.bfloat16)
    rr = r_ref[:, pl.ds(c * res_dim, res_dim)].astype(jnp.bfloat16)
    pre_t = jax.lax.dot_general(
        wt, rr,
        dimension_numbers=(((1,), (1,)), ((), ())),
        preferred_element_type=jnp.float32,
    )
    pre = pre_t.T
    pre = pre + u_ref[:, pl.ds(col, BI)] + BIAS
    r_slice = r_ref[:, pl.ds(col, BI)]
    o_ref[:, pl.ds(col, BI)] = LEAK * jnp.tanh(pre) + (1.0 - LEAK) * r_slice


def kernel(proj_vars, res_state, wr):
    seq, chunks, res_dim = proj_vars.shape
    flat = chunks * res_dim
    u = proj_vars.reshape(seq, flat)
    r = res_state.reshape(seq, flat)
    w = wr.reshape(flat, res_dim)
    n = flat // BI

    body = functools.partial(_esn_block, res_dim=res_dim)

    out = pl.pallas_call(
        body,
        grid=(n,),
        in_specs=[
            pl.BlockSpec((BI, res_dim), lambda i: (i, 0)),
            pl.BlockSpec((seq, flat), lambda i: (0, 0)),
            pl.BlockSpec((seq, flat), lambda i: (0, 0)),
        ],
        out_specs=pl.BlockSpec((seq, flat), lambda i: (0, 0)),
        out_shape=jax.ShapeDtypeStruct((seq, flat), jnp.float32),
        compiler_params=pltpu.CompilerParams(
            dimension_semantics=("arbitrary",),
        ),
    )(w, r, u)
    return out.reshape(seq, chunks, res_dim)
